# baseline (speedup 1.0000x reference)
import functools

import jax
import jax.numpy as jnp
from jax import lax
from jax.experimental import pallas as pl
from jax.experimental.pallas import tpu as pltpu
from jax.experimental.pallas import tpu_sc as plsc

L = 16
W = 128
H = 128
NV = W // L
IMG = H * W
G = 144
CH = 32
NCH = H // CH
NCORES = 2
NSUB = 16
NWORKERS = NCORES * NSUB
NIMG = 2 * 96
PER_W = NIMG // NWORKERS
NTAP = 9
TAPS = [(di, dj) for di in (-1, 0, 1) for dj in (-1, 0, 1)]


def _sc_body(s_hbm, so_hbm, img, tba, tbb, sema, semb):
    cid = lax.axis_index("c")
    sid = lax.axis_index("s")
    wid = sid * NCORES + cid

    lanes = lax.iota(jnp.int32, L)
    zeros = jnp.zeros((L,), jnp.float32)
    m_first = jnp.where(lanes > 0, 1.0, 0.0).astype(jnp.float32)
    m_last = jnp.where(lanes < L - 1, 1.0, 0.0).astype(jnp.float32)

    for i in range(G // L):
        img[pl.ds(i * L, L)] = zeros
        img[pl.ds(G + IMG + i * L, L)] = zeros

    def make_rows(tb):
        def do_rows(j, r0):
            h = r0 + j
            cbase = G + h * W
            cv = [img[pl.ds(cbase + L * v, L)] for v in range(NV)]
            slot = 0
            for t, (di, dj) in enumerate(TAPS):
                if t == 4:
                    continue
                sbase = G + (h + di) * W + dj
                for v in range(NV):
                    sv = img[pl.ds(sbase + L * v, L)]
                    p = sv * cv[v]
                    if dj == -1 and v == 0:
                        p = p * m_first
                    if dj == 1 and v == NV - 1:
                        p = p * m_last
                    tb[pl.ds(slot * CH * W + j * W + L * v, L)] = p
                slot += 1
            return r0

        return do_rows

    rows_a = make_rows(tba)
    rows_b = make_rows(tbb)

    def fire(tb, sem, o_hbm, n9, c):
        r0 = c * CH
        handles = []
        slot = 0
        for t in range(NTAP):
            if t == 4:
                src = img.at[pl.ds(G + r0 * W, CH * W)]
            else:
                src = tb.at[pl.ds(slot * CH * W, CH * W)]
                slot += 1
            handles.append(
                pltpu.async_copy(src, o_hbm.at[n9 + t, pl.ds(r0 * W, CH * W)], sem)
            )
        return handles

    def do_image(x_hbm, o_hbm, n):
        pltpu.sync_copy(x_hbm.at[n], img.at[pl.ds(G, IMG)])
        n9 = n * NTAP
        lax.fori_loop(0, CH, rows_a, 0 * CH)
        ha0 = fire(tba, sema, o_hbm, n9, 0)
        lax.fori_loop(0, CH, rows_b, 1 * CH)
        hb1 = fire(tbb, semb, o_hbm, n9, 1)
        for hd in ha0:
            hd.wait()
        lax.fori_loop(0, CH, rows_a, 2 * CH)
        ha2 = fire(tba, sema, o_hbm, n9, 2)
        for hd in hb1:
            hd.wait()
        lax.fori_loop(0, CH, rows_b, 3 * CH)
        hb3 = fire(tbb, semb, o_hbm, n9, 3)
        for hd in ha2:
            hd.wait()
        for hd in hb3:
            hd.wait()

    def s_loop(i, w):
        do_image(s_hbm, so_hbm, w * PER_W + i)
        return w

    lax.fori_loop(0, PER_W, s_loop, wid)


def _sc(x2):
    mesh = plsc.VectorSubcoreMesh(
        core_axis_name="c", subcore_axis_name="s",
        num_cores=NCORES, num_subcores=NSUB,
    )
    out = jax.ShapeDtypeStruct((NIMG * NTAP, IMG), jnp.float32)
    return pl.kernel(
        _sc_body,
        out_type=out,
        mesh=mesh,
        scratch_types=[
            pltpu.VMEM((2 * G + IMG,), jnp.float32),
            pltpu.VMEM(((NTAP - 1) * CH * W,), jnp.float32),
            pltpu.VMEM(((NTAP - 1) * CH * W,), jnp.float32),
            pltpu.SemaphoreType.DMA,
            pltpu.SemaphoreType.DMA,
        ],
        compiler_params=pltpu.CompilerParams(
            use_tc_tiling_on_sc=False, skip_device_barrier=True
        ),
    )(x2)


IPB = 16


def _tc_body(x_ref, o_ref):
    zrow = jnp.zeros((1, W), jnp.float32)
    r_ix = lax.broadcasted_iota(jnp.int32, (W, W), 0)
    c_ix = lax.broadcasted_iota(jnp.int32, (W, W), 1)
    s_m1 = (c_ix == r_ix + 1).astype(jnp.float32)
    s_p1 = (c_ix + 1 == r_ix).astype(jnp.float32)

    def mm(a, b):
        return jax.lax.dot_general(
            a, b, (((1,), (0,)), ((), ())),
            preferred_element_type=jnp.float32,
        )

    for b in range(IPB):
        x = x_ref[b]
        xc = {-1: mm(x, s_m1), 0: x, 1: mm(x, s_p1)}
        for t, (di, dj) in enumerate(TAPS):
            if t == 4:
                o_ref[b, t] = x
                continue
            xs = xc[dj]
            if di == -1:
                xs = jnp.concatenate([zrow, xs[:-1, :]], axis=0)
            elif di == 1:
                xs = jnp.concatenate([xs[1:, :], zrow], axis=0)
            o_ref[b, t] = xs * x


def _tc(x3):
    return pl.pallas_call(
        _tc_body,
        grid=(NIMG // IPB,),
        in_specs=[pl.BlockSpec((IPB, H, W), lambda i: (i, 0, 0))],
        out_specs=pl.BlockSpec((IPB, NTAP, H, W), lambda i: (i, 0, 0, 0)),
        out_shape=jax.ShapeDtypeStruct((NIMG, NTAP, H, W), jnp.float32),
        compiler_params=pltpu.CompilerParams(skip_device_barrier=True),
    )(x3)


@jax.jit
def _run(s2, cs3):
    return _sc(s2), _tc(cs3)


def kernel(s, cs):
    B, C = s.shape[0], s.shape[1]
    so, co = _run(s.reshape(NIMG, IMG), cs.reshape(NIMG, H, W))
    shape = (B, C, NTAP, H, W)
    return so.reshape(shape), co.reshape(shape)


# speedup vs baseline: 1.0314x; 1.0314x over previous
import functools

import jax
import jax.numpy as jnp
from jax import lax
from jax.experimental import pallas as pl
from jax.experimental.pallas import tpu as pltpu
from jax.experimental.pallas import tpu_sc as plsc

L = 16
W = 128
H = 128
NV = W // L
IMG = H * W
G = 144
CH = 32
NCH = H // CH
NCORES = 2
NSUB = 16
NWORKERS = NCORES * NSUB
NIMG = 2 * 96
PER_W = NIMG // NWORKERS
NTAP = 9
TAPS = [(di, dj) for di in (-1, 0, 1) for dj in (-1, 0, 1)]


def _sc_body(s_hbm, so_hbm, imga, imgb, tba, tbb, sema, semb, semi):
    cid = lax.axis_index("c")
    sid = lax.axis_index("s")
    wid = sid * NCORES + cid
    base_n = wid * PER_W

    lanes = lax.iota(jnp.int32, L)
    zeros = jnp.zeros((L,), jnp.float32)
    m_first = jnp.where(lanes > 0, 1.0, 0.0).astype(jnp.float32)
    m_last = jnp.where(lanes < L - 1, 1.0, 0.0).astype(jnp.float32)

    for img in (imga, imgb):
        for i in range(G // L):
            img[pl.ds(i * L, L)] = zeros
            img[pl.ds(G + IMG + i * L, L)] = zeros

    def make_rows(img, tb):
        def do_rows(j, r0):
            h = r0 + j
            cbase = G + h * W
            cv = [img[pl.ds(cbase + L * v, L)] for v in range(NV)]
            slot = 0
            for t, (di, dj) in enumerate(TAPS):
                if t == 4:
                    continue
                sbase = G + (h + di) * W + dj
                for v in range(NV):
                    sv = img[pl.ds(sbase + L * v, L)]
                    p = sv * cv[v]
                    if dj == -1 and v == 0:
                        p = p * m_first
                    if dj == 1 and v == NV - 1:
                        p = p * m_last
                    tb[pl.ds(slot * CH * W + j * W + L * v, L)] = p
                slot += 1
            return r0

        return do_rows

    rows_aa = make_rows(imga, tba)
    rows_ab = make_rows(imga, tbb)
    rows_ba = make_rows(imgb, tba)
    rows_bb = make_rows(imgb, tbb)

    def fire(img, tb, sem, o_hbm, n9, c):
        r0 = c * CH
        handles = []
        slot = 0
        for t in range(NTAP):
            if t == 4:
                src = img.at[pl.ds(G + r0 * W, CH * W)]
            else:
                src = tb.at[pl.ds(slot * CH * W, CH * W)]
                slot += 1
            handles.append(
                pltpu.async_copy(src, o_hbm.at[n9 + t, pl.ds(r0 * W, CH * W)], sem)
            )
        return handles

    def prefetch(x_hbm, img, n):
        return pltpu.async_copy(x_hbm.at[n], img.at[pl.ds(G, IMG)], semi)

    def do_image(img, rows_x, rows_y, o_hbm, n):
        n9 = n * NTAP
        lax.fori_loop(0, CH, rows_x, 0 * CH)
        h0 = fire(img, tba, sema, o_hbm, n9, 0)
        lax.fori_loop(0, CH, rows_y, 1 * CH)
        h1 = fire(img, tbb, semb, o_hbm, n9, 1)
        for hd in h0:
            hd.wait()
        lax.fori_loop(0, CH, rows_x, 2 * CH)
        h2 = fire(img, tba, sema, o_hbm, n9, 2)
        for hd in h1:
            hd.wait()
        lax.fori_loop(0, CH, rows_y, 3 * CH)
        h3 = fire(img, tbb, semb, o_hbm, n9, 3)
        for hd in h2:
            hd.wait()
        for hd in h3:
            hd.wait()

    prefetch(s_hbm, imga, base_n).wait()

    def pair(i, w):
        n = base_n + 2 * i
        pf_b = prefetch(s_hbm, imgb, n + 1)
        do_image(imga, rows_aa, rows_ab, so_hbm, n)
        pf_b.wait()
        pf_a = prefetch(s_hbm, imga, jnp.minimum(n + 2, base_n + PER_W - 1))
        do_image(imgb, rows_ba, rows_bb, so_hbm, n + 1)
        pf_a.wait()
        return w

    lax.fori_loop(0, PER_W // 2, pair, wid)


def _sc(x2):
    mesh = plsc.VectorSubcoreMesh(
        core_axis_name="c", subcore_axis_name="s",
        num_cores=NCORES, num_subcores=NSUB,
    )
    out = jax.ShapeDtypeStruct((NIMG * NTAP, IMG), jnp.float32)
    return pl.kernel(
        _sc_body,
        out_type=out,
        mesh=mesh,
        scratch_types=[
            pltpu.VMEM((2 * G + IMG,), jnp.float32),
            pltpu.VMEM((2 * G + IMG,), jnp.float32),
            pltpu.VMEM(((NTAP - 1) * CH * W,), jnp.float32),
            pltpu.VMEM(((NTAP - 1) * CH * W,), jnp.float32),
            pltpu.SemaphoreType.DMA,
            pltpu.SemaphoreType.DMA,
            pltpu.SemaphoreType.DMA,
        ],
        compiler_params=pltpu.CompilerParams(
            use_tc_tiling_on_sc=False, skip_device_barrier=True
        ),
    )(x2)


IPB = 16


def _tc_body(x_ref, o_ref):
    zrow = jnp.zeros((1, W), jnp.float32)
    r_ix = lax.broadcasted_iota(jnp.int32, (W, W), 0)
    c_ix = lax.broadcasted_iota(jnp.int32, (W, W), 1)
    s_m1 = (c_ix == r_ix + 1).astype(jnp.float32)
    s_p1 = (c_ix + 1 == r_ix).astype(jnp.float32)

    def mm(a, b):
        return jax.lax.dot_general(
            a, b, (((1,), (0,)), ((), ())),
            preferred_element_type=jnp.float32,
        )

    for b in range(IPB):
        x = x_ref[b]
        xc = {-1: mm(x, s_m1), 0: x, 1: mm(x, s_p1)}
        for t, (di, dj) in enumerate(TAPS):
            if t == 4:
                o_ref[b, t] = x
                continue
            xs = xc[dj]
            if di == -1:
                xs = jnp.concatenate([zrow, xs[:-1, :]], axis=0)
            elif di == 1:
                xs = jnp.concatenate([xs[1:, :], zrow], axis=0)
            o_ref[b, t] = xs * x


def _tc(x3):
    return pl.pallas_call(
        _tc_body,
        grid=(NIMG // IPB,),
        in_specs=[pl.BlockSpec((IPB, H, W), lambda i: (i, 0, 0))],
        out_specs=pl.BlockSpec((IPB, NTAP, H, W), lambda i: (i, 0, 0, 0)),
        out_shape=jax.ShapeDtypeStruct((NIMG, NTAP, H, W), jnp.float32),
        compiler_params=pltpu.CompilerParams(skip_device_barrier=True),
    )(x3)


@jax.jit
def _run(s2, cs3):
    return _sc(s2), _tc(cs3)


def kernel(s, cs):
    B, C = s.shape[0], s.shape[1]
    so, co = _run(s.reshape(NIMG, IMG), cs.reshape(NIMG, H, W))
    shape = (B, C, NTAP, H, W)
    return so.reshape(shape), co.reshape(shape)
